# baseline (speedup 1.0000x reference)
import jax
import jax.numpy as jnp
from jax import lax
from jax.experimental import pallas as pl
from jax.experimental.pallas import tpu as pltpu
from jax.experimental.pallas import tpu_sc as plsc

_B = 16384
_D = 128
_K = _D // 16
_NC = 2
_NS = 16
_NW = _NC * _NS
_RPW = _B // _NW
_C = 64
_NCH = _RPW // _C
_CATA = 2 * _C
_CATB = 3 * _C
_VA = 125
_VB = 2157
_OFFA = (0, 25)
_OFFB = (0, 400, 1100)


def _rsqrt_nr(x):
    xi = lax.bitcast_convert_type(x, jnp.int32)
    yi = jnp.int32(0x5F3759DF) - lax.shift_right_logical(xi, 1)
    y = lax.bitcast_convert_type(yi, jnp.float32)
    hx = x * jnp.float32(0.5)
    for _ in range(2):
        y = y * (jnp.float32(1.5) - hx * y * y)
    return y


def _splat(s):
    return lax.broadcast_in_dim(s, (16,), ())


def _body(i2, i3, i4, i5, i6, tca, tcb, wb,
          out_hbm,
          ix0, ix1, ix2, ix3, ix4, ixa, ixb,
          tsh, ga0, ga1, gb0, gb1, out_v, wb_v,
          semA0, semA1, semB0, semB1):
    sid = lax.axis_index("s")
    wid = sid * _NC + lax.axis_index("c")
    base = wid * _RPW

    @pl.when(sid == 0)
    def _():
        pltpu.sync_copy(tcb, tsh)

    idx_refs = (ix0, ix1, ix2, ix3, ix4)
    for idx_hbm, idx_v in zip((i2, i3, i4, i5, i6), idx_refs):
        pltpu.sync_copy(idx_hbm.at[pl.ds(base, _RPW)], idx_v)
    pltpu.sync_copy(wb, wb_v)

    for l in range(5):
        part = l < 2
        cat = _CATA if part else _CATB
        dst_ref = ixa if part else ixb
        off = jnp.full((16,), (_OFFA + _OFFB)[l], jnp.int32)
        lb = (l if part else l - 2) * _C
        src = idx_refs[l]

        def mk_cat(i, off=off, src=src, cat=cat, dst_ref=dst_ref, lb=lb):
            ch = i // 4
            q = i % 4
            dst = pl.multiple_of(ch * cat + lb + q * 16, 16)
            sp = pl.multiple_of(i * 16, 16)
            dst_ref[pl.ds(dst, 16)] = src[pl.ds(sp, 16)] + off

        plsc.parallel_loop(0, _RPW // 16)(mk_cat)

    gas = (ga0, ga1)
    gbs = (gb0, gb1)
    semsA = (semA0, semA1)
    semsB = (semB0, semB1)

    w_regs = [wb_v[pl.ds(k * 16, 16)] for k in range(_K)]
    b_splat = _splat(wb_v[pl.ds(_D, 16)][0])
    lane0 = lax.broadcasted_iota(jnp.int32, (16,), 0) == 0

    def issue(ch):
        p = ch % 2
        ca = pltpu.async_copy(
            tca.at[ixa.at[pl.ds(ch * _CATA, _CATA)]], gas[p], semsA[p])
        cb_ = pltpu.async_copy(
            tsh.at[ixb.at[pl.ds(ch * _CATB, _CATB)]], gbs[p], semsB[p])
        return (ca, cb_)

    plsc.subcore_barrier()
    pending = issue(0)
    for ch in range(_NCH):
        for c in pending:
            c.wait()
        pending = issue(ch + 1) if ch + 1 < _NCH else ()
        ga = gas[ch % 2]
        gb = gbs[ch % 2]
        cb = ch * _C

        def row_body(r, _, ga=ga, gb=gb, cb=cb):
            u = [ga[r, pl.ds(k * 16, 16)] for k in range(_K)]
            acc = u[0] * u[0]
            for k in range(1, _K):
                acc = acc + u[k] * u[k]
            y = _rsqrt_nr(_splat(jnp.sum(acc)))
            for l in range(1, 5):
                g = ga if l < 2 else gb
                row = (l if l < 2 else l - 2) * _C + r
                u = [y * u[k] + g[row, pl.ds(k * 16, 16)] for k in range(_K)]
                acc = u[0] * u[0]
                for k in range(1, _K):
                    acc = acc + u[k] * u[k]
                y = _rsqrt_nr(_splat(jnp.sum(acc)))
            dotv = u[0] * w_regs[0]
            for k in range(1, _K):
                dotv = dotv + u[k] * w_regs[k]
            row_out = y * _splat(jnp.sum(dotv)) + b_splat
            plsc.store_scatter(out_v, [jnp.full((16,), cb + r, jnp.int32)],
                               row_out, mask=lane0)
            return 0

        plsc.parallel_loop(0, _C, carry=jnp.int32(0))(row_body)

    pltpu.sync_copy(out_v, out_hbm.at[pl.ds(base, _RPW)])


def kernel(naics_2_digit, naics_3_digit, naics_4_digit, naics_5_digit, naics_6_digit,
           table2, delta3, delta4, delta5, delta6, W, b):
    tca = jnp.concatenate([table2, delta3], axis=0)
    tcb = jnp.concatenate([delta4, delta5, delta6], axis=0)
    wb = jnp.concatenate([W.reshape(_D), b, jnp.zeros((15,), jnp.float32)])
    mesh = plsc.VectorSubcoreMesh(core_axis_name="c", subcore_axis_name="s")
    scratch = [pltpu.VMEM((_RPW,), jnp.int32)] * 5 + [
        pltpu.VMEM((_NCH * _CATA,), jnp.int32),
        pltpu.VMEM((_NCH * _CATB,), jnp.int32),
        pltpu.VMEM_SHARED((_VB, _D), jnp.float32),
        pltpu.VMEM((_CATA, _D), jnp.float32),
        pltpu.VMEM((_CATA, _D), jnp.float32),
        pltpu.VMEM((_CATB, _D), jnp.float32),
        pltpu.VMEM((_CATB, _D), jnp.float32),
        pltpu.VMEM((_RPW,), jnp.float32),
        pltpu.VMEM((_D + 16,), jnp.float32),
        pltpu.SemaphoreType.DMA,
        pltpu.SemaphoreType.DMA,
        pltpu.SemaphoreType.DMA,
        pltpu.SemaphoreType.DMA,
    ]
    call = pl.kernel(
        _body,
        out_type=jax.ShapeDtypeStruct((_B,), jnp.float32),
        mesh=mesh,
        scratch_types=scratch,
        compiler_params=pltpu.CompilerParams(needs_layout_passes=False),
    )
    out = call(naics_2_digit, naics_3_digit, naics_4_digit, naics_5_digit,
               naics_6_digit, tca, tcb, wb)
    return out.reshape(_B, 1)
